# Initial kernel scaffold; baseline (speedup 1.0000x reference)
import jax
import jax.numpy as jnp
from jax.experimental import pallas as pl


def kernel(images, w1, b1, w2, b2, w3, b3, w_rpn, b_rpn, w_cls, b_cls, w_reg, b_reg):
    raise NotImplementedError("write your pallas kernel here")


# speedup vs baseline: 1.0004x; 1.0004x over previous
import functools

import jax
import jax.numpy as jnp
from jax.experimental import pallas as pl

IMG = 512
NUM_ANCHOR_TYPES = 15
PRE_NMS_TOPK = 2000
POST_NMS_KEEP = 300
IOU_THRESH = 0.7


def _conv2d(x, w, b, stride=1):
    out = jax.lax.conv_general_dilated(
        x, w, (stride, stride), 'SAME',
        dimension_numbers=('NCHW', 'OIHW', 'NCHW'))
    return out + b[None, :, None, None]


def _generate_anchors(fh, fw, stride):
    sizes = jnp.array([32.0, 64.0, 128.0, 256.0, 512.0], jnp.float32)
    ratios = jnp.array([0.5, 1.0, 2.0], jnp.float32)
    h_r = jnp.sqrt(ratios)
    w_r = 1.0 / h_r
    ws = (w_r[:, None] * sizes[None, :]).reshape(-1)
    hs = (h_r[:, None] * sizes[None, :]).reshape(-1)
    base = jnp.stack([-ws, -hs, ws, hs], axis=1) / 2.0
    sx = (jnp.arange(fw, dtype=jnp.float32) + 0.5) * stride
    sy = (jnp.arange(fh, dtype=jnp.float32) + 0.5) * stride
    yy, xx = jnp.meshgrid(sy, sx, indexing='ij')
    shifts = jnp.stack([xx.reshape(-1), yy.reshape(-1), xx.reshape(-1),
                        yy.reshape(-1)], axis=1)
    return (shifts[:, None, :] + base[None, :, :]).reshape(-1, 4)


def _decode_kernel(anchors_ref, deltas_ref, out_ref):
    a = anchors_ref[...]
    d = deltas_ref[...]
    wa = a[2] - a[0]
    ha = a[3] - a[1]
    cxa = a[0] + 0.5 * wa
    cya = a[1] + 0.5 * ha
    dx, dy = d[0], d[1]
    dw = jnp.clip(d[2], -4.0, 4.0)
    dh = jnp.clip(d[3], -4.0, 4.0)
    cx = dx * wa + cxa
    cy = dy * ha + cya
    w = jnp.exp(dw) * wa
    h = jnp.exp(dh) * ha
    boxes = jnp.stack([cx - 0.5 * w, cy - 0.5 * h, cx + 0.5 * w,
                       cy + 0.5 * h], axis=0)
    out_ref[...] = jnp.clip(boxes, 0.0, float(IMG))


def _decode_boxes_pallas(anchors, deltas):
    n = anchors.shape[0]
    out = pl.pallas_call(
        _decode_kernel,
        out_shape=jax.ShapeDtypeStruct((4, n), jnp.float32),
    )(anchors.T, deltas.T)
    return out.T


def _box_iou(a, b):
    area_a = (a[:, 2] - a[:, 0]) * (a[:, 3] - a[:, 1])
    area_b = (b[:, 2] - b[:, 0]) * (b[:, 3] - b[:, 1])
    lt = jnp.maximum(a[:, None, :2], b[None, :, :2])
    rb = jnp.minimum(a[:, None, 2:], b[None, :, 2:])
    wh = jnp.clip(rb - lt, 0.0, None)
    inter = wh[..., 0] * wh[..., 1]
    return inter / (area_a[:, None] + area_b[None, :] - inter + 1e-9)


def _nms_fixed(boxes, scores, iou_thresh, keep_k):
    n = boxes.shape[0]
    order = jnp.argsort(-scores)
    boxes_s = jnp.take(boxes, order, axis=0)
    scores_s = jnp.take(scores, order)
    iou = _box_iou(boxes_s, boxes_s)
    suppress_mat = iou > iou_thresh
    idx = jnp.arange(n)

    def body(keep, i):
        alive = keep[i]
        sup = suppress_mat[i] & (idx > i) & alive
        keep = keep & (~sup)
        return keep, None

    keep0 = jnp.ones((n,), dtype=bool)
    keep, _ = jax.lax.scan(body, keep0, jnp.arange(n))
    kept_scores = jnp.where(keep, scores_s, -1.0)
    top_v, top_i = jax.lax.top_k(kept_scores, keep_k)
    return jnp.take(boxes_s, top_i, axis=0), top_v


@jax.jit
def kernel(images, w1, b1, w2, b2, w3, b3, w_rpn, b_rpn, w_cls, b_cls,
           w_reg, b_reg):
    x = jax.nn.relu(_conv2d(images, w1, b1, 2))
    x = jax.nn.relu(_conv2d(x, w2, b2, 2))
    feat = jax.nn.relu(_conv2d(x, w3, b3, 2))
    t = jax.nn.relu(_conv2d(feat, w_rpn, b_rpn, 1))
    logits = _conv2d(t, w_cls, b_cls, 1)
    deltas = _conv2d(t, w_reg, b_reg, 1)
    A = NUM_ANCHOR_TYPES
    fh, fw = logits.shape[2], logits.shape[3]
    scores = jax.nn.sigmoid(jnp.transpose(logits[0], (1, 2, 0)).reshape(-1))
    d = jnp.transpose(deltas[0].reshape(A, 4, fh, fw), (2, 3, 0, 1)).reshape(-1, 4)
    anchors = _generate_anchors(fh, fw, float(IMG) / fh)
    boxes = _decode_boxes_pallas(anchors, d)
    top_s, top_i = jax.lax.top_k(scores, PRE_NMS_TOPK)
    pb = jnp.take(boxes, top_i, axis=0)
    return _nms_fixed(pb, top_s, IOU_THRESH, POST_NMS_KEEP)


# speedup vs baseline: 19.0499x; 19.0417x over previous
import functools

import jax
import jax.numpy as jnp
from jax.experimental import pallas as pl
from jax.experimental.pallas import tpu as pltpu

IMG = 512
NUM_ANCHOR_TYPES = 15
PRE_NMS_TOPK = 2000
POST_NMS_KEEP = 300
IOU_THRESH = 0.7


def _conv2d(x, w, b, stride=1):
    out = jax.lax.conv_general_dilated(
        x, w, (stride, stride), 'SAME',
        dimension_numbers=('NCHW', 'OIHW', 'NCHW'))
    return out + b[None, :, None, None]


def _generate_anchors(fh, fw, stride):
    sizes = jnp.array([32.0, 64.0, 128.0, 256.0, 512.0], jnp.float32)
    ratios = jnp.array([0.5, 1.0, 2.0], jnp.float32)
    h_r = jnp.sqrt(ratios)
    w_r = 1.0 / h_r
    ws = (w_r[:, None] * sizes[None, :]).reshape(-1)
    hs = (h_r[:, None] * sizes[None, :]).reshape(-1)
    base = jnp.stack([-ws, -hs, ws, hs], axis=1) / 2.0
    sx = (jnp.arange(fw, dtype=jnp.float32) + 0.5) * stride
    sy = (jnp.arange(fh, dtype=jnp.float32) + 0.5) * stride
    yy, xx = jnp.meshgrid(sy, sx, indexing='ij')
    shifts = jnp.stack([xx.reshape(-1), yy.reshape(-1), xx.reshape(-1),
                        yy.reshape(-1)], axis=1)
    return (shifts[:, None, :] + base[None, :, :]).reshape(-1, 4)


def _decode_kernel(anchors_ref, deltas_ref, out_ref):
    a = anchors_ref[...]
    d = deltas_ref[...]
    wa = a[2] - a[0]
    ha = a[3] - a[1]
    cxa = a[0] + 0.5 * wa
    cya = a[1] + 0.5 * ha
    dx, dy = d[0], d[1]
    dw = jnp.clip(d[2], -4.0, 4.0)
    dh = jnp.clip(d[3], -4.0, 4.0)
    cx = dx * wa + cxa
    cy = dy * ha + cya
    w = jnp.exp(dw) * wa
    h = jnp.exp(dh) * ha
    boxes = jnp.stack([cx - 0.5 * w, cy - 0.5 * h, cx + 0.5 * w,
                       cy + 0.5 * h], axis=0)
    out_ref[...] = jnp.clip(boxes, 0.0, float(IMG))


def _decode_boxes_pallas(anchors, deltas):
    n = anchors.shape[0]
    out = pl.pallas_call(
        _decode_kernel,
        out_shape=jax.ShapeDtypeStruct((4, n), jnp.float32),
    )(anchors.T, deltas.T)
    return out.T


_NMS_N = 2048
_NMS_CHUNK = 256


def _lane_cumsum(x, n):
    d = 1
    while d < n:
        shifted = jnp.concatenate(
            [jnp.zeros((1, d), jnp.float32), x[:, :n - d]], axis=1)
        x = x + shifted
        d *= 2
    return x


def _nms_pallas_kernel(boxes_r_ref, boxes_c_ref, scores_ref,
                       out_boxes_ref, out_scores_ref, s_ref, keep_ref):
    n = _NMS_N
    ch = _NMS_CHUNK
    bx1 = boxes_r_ref[0:1, :]
    by1 = boxes_r_ref[1:2, :]
    bx2 = boxes_r_ref[2:3, :]
    by2 = boxes_r_ref[3:4, :]
    area_b = (bx2 - bx1) * (by2 - by1)

    for k in range(n // ch):
        r0 = k * ch
        ax1 = boxes_c_ref[pl.ds(r0, ch), 0:1]
        ay1 = boxes_c_ref[pl.ds(r0, ch), 1:2]
        ax2 = boxes_c_ref[pl.ds(r0, ch), 2:3]
        ay2 = boxes_c_ref[pl.ds(r0, ch), 3:4]
        area_a = (ax2 - ax1) * (ay2 - ay1)
        w = jnp.clip(jnp.minimum(ax2, bx2) - jnp.maximum(ax1, bx1), 0.0, None)
        h = jnp.clip(jnp.minimum(ay2, by2) - jnp.maximum(ay1, by1), 0.0, None)
        inter = w * h
        iou = inter / (area_a + area_b - inter + 1e-9)
        iidx = jax.lax.broadcasted_iota(jnp.int32, (ch, n), 0) + r0
        jidx = jax.lax.broadcasted_iota(jnp.int32, (ch, n), 1)
        s = jnp.where((iou > IOU_THRESH) & (jidx > iidx), 1.0, 0.0)
        s_ref[pl.ds(r0, ch), :] = s

    lane = jax.lax.broadcasted_iota(jnp.int32, (1, n), 1)
    keep_init = jnp.where(lane < PRE_NMS_TOPK, 1.0, 0.0)

    def body(t, keep):
        base = pl.multiple_of(t * 8, 8)
        rows8 = s_ref[pl.ds(base, 8), :]
        for r in range(8):
            i = t * 8 + r
            onehot = jnp.where(lane == i, 1.0, 0.0)
            alive = jnp.sum(keep * onehot)
            keep = keep * (1.0 - rows8[r:r + 1, :] * alive)
        return keep

    keep = jax.lax.fori_loop(0, PRE_NMS_TOPK // 8, body, keep_init)
    keep_ref[...] = keep

    keep = keep_ref[...]
    scores = scores_ref[...]
    kept_scores = scores * keep - (1.0 - keep)
    csum_keep = _lane_cumsum(keep, n)
    csum_not = _lane_cumsum(1.0 - keep, n)
    total_kept = csum_keep[0:1, n - 1:n]
    pos = jnp.where(keep > 0.0, csum_keep - 1.0, total_kept + csum_not - 1.0)

    jpos = jax.lax.broadcasted_iota(
        jnp.int32, (POST_NMS_KEEP, n), 0).astype(jnp.float32)
    eq = jnp.where(jpos == pos, 1.0, 0.0)

    rx1 = jnp.sum(eq * bx1, axis=1, keepdims=True)
    ry1 = jnp.sum(eq * by1, axis=1, keepdims=True)
    rx2 = jnp.sum(eq * bx2, axis=1, keepdims=True)
    ry2 = jnp.sum(eq * by2, axis=1, keepdims=True)
    out_boxes_ref[...] = jnp.concatenate([rx1, ry1, rx2, ry2], axis=1)
    out_scores_ref[...] = jnp.sum(eq * kept_scores, axis=1, keepdims=True)


def _nms_pallas(pb, top_s):
    n = _NMS_N
    k = pb.shape[0]
    boxes_c = jnp.zeros((n, 4), jnp.float32).at[:k].set(pb)
    boxes_r = boxes_c.T
    scores = jnp.full((1, n), -1.0, jnp.float32).at[0, :k].set(top_s)
    out_boxes, out_scores = pl.pallas_call(
        _nms_pallas_kernel,
        out_shape=(
            jax.ShapeDtypeStruct((POST_NMS_KEEP, 4), jnp.float32),
            jax.ShapeDtypeStruct((POST_NMS_KEEP, 1), jnp.float32),
        ),
        scratch_shapes=[
            pltpu.VMEM((n, n), jnp.float32),
            pltpu.VMEM((1, n), jnp.float32),
        ],
    )(boxes_r, boxes_c, scores)
    return out_boxes, out_scores[:, 0]


def _box_iou(a, b):
    area_a = (a[:, 2] - a[:, 0]) * (a[:, 3] - a[:, 1])
    area_b = (b[:, 2] - b[:, 0]) * (b[:, 3] - b[:, 1])
    lt = jnp.maximum(a[:, None, :2], b[None, :, :2])
    rb = jnp.minimum(a[:, None, 2:], b[None, :, 2:])
    wh = jnp.clip(rb - lt, 0.0, None)
    inter = wh[..., 0] * wh[..., 1]
    return inter / (area_a[:, None] + area_b[None, :] - inter + 1e-9)


def _nms_fixed(boxes, scores, iou_thresh, keep_k):
    n = boxes.shape[0]
    order = jnp.argsort(-scores)
    boxes_s = jnp.take(boxes, order, axis=0)
    scores_s = jnp.take(scores, order)
    iou = _box_iou(boxes_s, boxes_s)
    suppress_mat = iou > iou_thresh
    idx = jnp.arange(n)

    def body(keep, i):
        alive = keep[i]
        sup = suppress_mat[i] & (idx > i) & alive
        keep = keep & (~sup)
        return keep, None

    keep0 = jnp.ones((n,), dtype=bool)
    keep, _ = jax.lax.scan(body, keep0, jnp.arange(n))
    kept_scores = jnp.where(keep, scores_s, -1.0)
    top_v, top_i = jax.lax.top_k(kept_scores, keep_k)
    return jnp.take(boxes_s, top_i, axis=0), top_v


@jax.jit
def kernel(images, w1, b1, w2, b2, w3, b3, w_rpn, b_rpn, w_cls, b_cls,
           w_reg, b_reg):
    x = jax.nn.relu(_conv2d(images, w1, b1, 2))
    x = jax.nn.relu(_conv2d(x, w2, b2, 2))
    feat = jax.nn.relu(_conv2d(x, w3, b3, 2))
    t = jax.nn.relu(_conv2d(feat, w_rpn, b_rpn, 1))
    logits = _conv2d(t, w_cls, b_cls, 1)
    deltas = _conv2d(t, w_reg, b_reg, 1)
    A = NUM_ANCHOR_TYPES
    fh, fw = logits.shape[2], logits.shape[3]
    scores = jax.nn.sigmoid(jnp.transpose(logits[0], (1, 2, 0)).reshape(-1))
    d = jnp.transpose(deltas[0].reshape(A, 4, fh, fw), (2, 3, 0, 1)).reshape(-1, 4)
    anchors = _generate_anchors(fh, fw, float(IMG) / fh)
    boxes = _decode_boxes_pallas(anchors, d)
    top_s, top_i = jax.lax.top_k(scores, PRE_NMS_TOPK)
    pb = jnp.take(boxes, top_i, axis=0)
    return _nms_pallas(pb, top_s)
